# baseline (speedup 1.0000x reference)
import functools

import jax
import jax.numpy as jnp
from jax import lax
from jax.experimental import pallas as pl
from jax.experimental.pallas import tpu as pltpu
from jax.experimental.pallas import tpu_sc as plsc

NUM_ENT = 100000
NUM_REL = 500
DIM = 32
B = 1024
K = 8 * DIM
TN = 1024
NBLK = (NUM_ENT + TN - 1) // TN

NC = 2
NS = 16
NW = NC * NS
BPW = B // NW


def _octonion_mul(O1, O2):
    x0, x1, x2, x3, x4, x5, x6, x7 = O1
    y0, y1, y2, y3, y4, y5, y6, y7 = O2
    e0 = x0*y0 - x1*y1 - x2*y2 - x3*y3 - x4*y4 - x5*y5 - x6*y6 - x7*y7
    e1 = x0*y1 + x1*y0 + x2*y3 - x3*y2 + x4*y5 - x5*y4 - x6*y7 + x7*y6
    e2 = x0*y2 - x1*y3 + x2*y0 + x3*y1 + x4*y6 + x5*y7 - x6*y4 - x7*y5
    e3 = x0*y3 + x1*y2 - x2*y1 + x3*y0 + x4*y7 - x5*y6 + x6*y5 - x7*y4
    e4 = x0*y4 - x1*y5 - x2*y6 - x3*y7 + x4*y0 + x5*y1 + x6*y2 + x7*y3
    e5 = x0*y5 + x1*y4 - x2*y7 + x3*y6 - x4*y1 + x5*y0 - x6*y3 + x7*y2
    e6 = x0*y6 + x1*y7 + x2*y4 - x3*y5 - x4*y2 + x5*y3 + x6*y0 - x7*y1
    e7 = x0*y7 - x1*y6 + x2*y5 + x3*y4 - x4*y3 - x5*y2 + x6*y1 + x7*y0
    return (e0, e1, e2, e3, e4, e5, e6, e7)


def _make_gather_sc():
    mesh = plsc.VectorSubcoreMesh(core_axis_name="c", subcore_axis_name="s")

    @functools.partial(
        pl.kernel, mesh=mesh,
        out_type=[jax.ShapeDtypeStruct((B, K), jnp.float32),
                  jax.ShapeDtypeStruct((B, K), jnp.float32)],
        scratch_types=[
            pltpu.VMEM((BPW,), jnp.int32),
            pltpu.VMEM((BPW,), jnp.int32),
            pltpu.VMEM((BPW, K), jnp.float32),
            pltpu.VMEM((BPW, K), jnp.float32),
            pltpu.SemaphoreType.DMA,
            pltpu.SemaphoreType.DMA,
        ],
    )
    def gather(e1_hbm, rel_hbm, ecat_hbm, rcat_hbm, hout, rout,
               eidx, ridx, hrows, rrows, sem_h, sem_r):
        wid = lax.axis_index("s") * NC + lax.axis_index("c")
        base = wid * BPW
        pltpu.sync_copy(e1_hbm.at[pl.ds(base, BPW)], eidx)
        pltpu.sync_copy(rel_hbm.at[pl.ds(base, BPW)], ridx)
        ch = pltpu.async_copy(ecat_hbm.at[eidx], hrows, sem_h)
        cr = pltpu.async_copy(rcat_hbm.at[ridx], rrows, sem_r)
        ch.wait()
        pltpu.sync_copy(hrows, hout.at[pl.ds(base, BPW)])
        cr.wait()
        pltpu.sync_copy(rrows, rout.at[pl.ds(base, BPW)])

    return gather


def _score_kernel(hrows, rrows, ecat, out_ref, hcat):
    @pl.when(pl.program_id(0) == 0)
    def _build_h():
        ys = [rrows[:, DIM * i:DIM * (i + 1)] for i in range(8)]
        inv = lax.rsqrt(sum(y * y for y in ys))
        ys = [y * inv for y in ys]
        xs = [hrows[:, DIM * i:DIM * (i + 1)] for i in range(8)]
        es = _octonion_mul(xs, ys)
        hcat[...] = jnp.concatenate(es, axis=1)

    acc = lax.dot_general(
        hcat[...], ecat[...],
        (((1,), (1,)), ((), ())),
        preferred_element_type=jnp.float32)
    out_ref[...] = jax.nn.sigmoid(acc)


def _score(hrows, rrows, ecat, interpret=False):
    full = pl.BlockSpec((B, K), lambda n: (0, 0))
    eblk = pl.BlockSpec((TN, K), lambda n: (n, 0))
    return pl.pallas_call(
        _score_kernel,
        grid=(NBLK,),
        in_specs=[full, full, eblk],
        out_specs=pl.BlockSpec((B, TN), lambda n: (0, n)),
        out_shape=jax.ShapeDtypeStruct((B, NUM_ENT), jnp.float32),
        scratch_shapes=[pltpu.VMEM((B, K), jnp.float32)],
        compiler_params=pltpu.CompilerParams(
            dimension_semantics=("arbitrary",)),
        interpret=interpret,
    )(hrows, rrows, ecat)


def kernel(E0, E1, E2, E3, E4, E5, E6, E7,
           R0, R1, R2, R3, R4, R5, R6, R7, e1_idx, rel_idx):
    ents = (E0, E1, E2, E3, E4, E5, E6, E7)
    rel_tables = (R0, R1, R2, R3, R4, R5, R6, R7)
    ecat = jnp.concatenate(ents, axis=1)
    rcat = jnp.concatenate(rel_tables, axis=1)
    hrows, rrows = _make_gather_sc()(e1_idx, rel_idx, ecat, rcat)
    return _score(hrows, rrows, ecat)


# speedup vs baseline: 1.0469x; 1.0469x over previous
import functools

import jax
import jax.numpy as jnp
from jax import lax
from jax.experimental import pallas as pl
from jax.experimental.pallas import tpu as pltpu
from jax.experimental.pallas import tpu_sc as plsc

NUM_ENT = 100000
NUM_REL = 500
DIM = 32
B = 1024
K = 8 * DIM
TN = 2048
NBLK = (NUM_ENT + TN - 1) // TN

NC = 2
NS = 16
NW = NC * NS
BPW = B // NW


def _octonion_mul(O1, O2):
    x0, x1, x2, x3, x4, x5, x6, x7 = O1
    y0, y1, y2, y3, y4, y5, y6, y7 = O2
    e0 = x0*y0 - x1*y1 - x2*y2 - x3*y3 - x4*y4 - x5*y5 - x6*y6 - x7*y7
    e1 = x0*y1 + x1*y0 + x2*y3 - x3*y2 + x4*y5 - x5*y4 - x6*y7 + x7*y6
    e2 = x0*y2 - x1*y3 + x2*y0 + x3*y1 + x4*y6 + x5*y7 - x6*y4 - x7*y5
    e3 = x0*y3 + x1*y2 - x2*y1 + x3*y0 + x4*y7 - x5*y6 + x6*y5 - x7*y4
    e4 = x0*y4 - x1*y5 - x2*y6 - x3*y7 + x4*y0 + x5*y1 + x6*y2 + x7*y3
    e5 = x0*y5 + x1*y4 - x2*y7 + x3*y6 - x4*y1 + x5*y0 - x6*y3 + x7*y2
    e6 = x0*y6 + x1*y7 + x2*y4 - x3*y5 - x4*y2 + x5*y3 + x6*y0 - x7*y1
    e7 = x0*y7 - x1*y6 + x2*y5 + x3*y4 - x4*y3 - x5*y2 + x6*y1 + x7*y0
    return (e0, e1, e2, e3, e4, e5, e6, e7)


def _make_gather_sc():
    mesh = plsc.VectorSubcoreMesh(core_axis_name="c", subcore_axis_name="s")

    @functools.partial(
        pl.kernel, mesh=mesh,
        out_type=[jax.ShapeDtypeStruct((B, K), jnp.float32),
                  jax.ShapeDtypeStruct((B, K), jnp.float32)],
        scratch_types=[
            pltpu.VMEM((BPW,), jnp.int32),
            pltpu.VMEM((BPW,), jnp.int32),
            pltpu.VMEM((BPW, K), jnp.float32),
            pltpu.VMEM((BPW, K), jnp.float32),
            pltpu.SemaphoreType.DMA,
            pltpu.SemaphoreType.DMA,
        ],
    )
    def gather(e1_hbm, rel_hbm, ecat_hbm, rcat_hbm, hout, rout,
               eidx, ridx, hrows, rrows, sem_h, sem_r):
        wid = lax.axis_index("s") * NC + lax.axis_index("c")
        base = wid * BPW
        pltpu.sync_copy(e1_hbm.at[pl.ds(base, BPW)], eidx)
        pltpu.sync_copy(rel_hbm.at[pl.ds(base, BPW)], ridx)
        ch = pltpu.async_copy(ecat_hbm.at[eidx], hrows, sem_h)
        cr = pltpu.async_copy(rcat_hbm.at[ridx], rrows, sem_r)
        ch.wait()
        pltpu.sync_copy(hrows, hout.at[pl.ds(base, BPW)])
        cr.wait()
        pltpu.sync_copy(rrows, rout.at[pl.ds(base, BPW)])

    return gather


def _score_kernel(hrows, rrows, ecat, out_ref, hcat):
    @pl.when(pl.program_id(0) == 0)
    def _build_h():
        ys = [rrows[:, DIM * i:DIM * (i + 1)] for i in range(8)]
        inv = lax.rsqrt(sum(y * y for y in ys))
        ys = [y * inv for y in ys]
        xs = [hrows[:, DIM * i:DIM * (i + 1)] for i in range(8)]
        es = _octonion_mul(xs, ys)
        hcat[...] = jnp.concatenate(es, axis=1)

    acc = lax.dot_general(
        hcat[...], ecat[...],
        (((1,), (1,)), ((), ())),
        preferred_element_type=jnp.float32)
    out_ref[...] = jax.nn.sigmoid(acc)


def _score(hrows, rrows, ecat, interpret=False):
    full = pl.BlockSpec((B, K), lambda n: (0, 0))
    eblk = pl.BlockSpec((TN, K), lambda n: (n, 0))
    return pl.pallas_call(
        _score_kernel,
        grid=(NBLK,),
        in_specs=[full, full, eblk],
        out_specs=pl.BlockSpec((B, TN), lambda n: (0, n)),
        out_shape=jax.ShapeDtypeStruct((B, NUM_ENT), jnp.float32),
        scratch_shapes=[pltpu.VMEM((B, K), jnp.float32)],
        compiler_params=pltpu.CompilerParams(
            dimension_semantics=("arbitrary",)),
        interpret=interpret,
    )(hrows, rrows, ecat)


def kernel(E0, E1, E2, E3, E4, E5, E6, E7,
           R0, R1, R2, R3, R4, R5, R6, R7, e1_idx, rel_idx):
    ents = (E0, E1, E2, E3, E4, E5, E6, E7)
    rel_tables = (R0, R1, R2, R3, R4, R5, R6, R7)
    ecat = jnp.concatenate(ents, axis=1)
    rcat = jnp.concatenate(rel_tables, axis=1)
    hrows, rrows = _make_gather_sc()(e1_idx, rel_idx, ecat, rcat)
    return _score(hrows, rrows, ecat)
